# baseline (speedup 1.0000x reference)
import functools

import jax
import jax.numpy as jnp
from jax import lax
from jax.experimental import pallas as pl
from jax.experimental.pallas import tpu as pltpu
from jax.experimental.pallas import tpu_sc as plsc

D = 128
B = 4096 * 50
NC, NS = 2, 16
NW = NC * NS
B_PER_W = B // NW
C = 128
NCHUNK = B_PER_W // C
NBUF = 5
NGROUP = NCHUNK // NBUF

_mesh = plsc.VectorSubcoreMesh(core_axis_name="c", subcore_axis_name="s")


@functools.partial(
    pl.kernel,
    mesh=_mesh,
    out_type=jax.ShapeDtypeStruct((B, D), jnp.float32),
    scratch_types=[
        pltpu.VMEM((B_PER_W,), jnp.int32),
        pltpu.VMEM((NBUF, C, D), jnp.float32),
    ]
    + [pltpu.SemaphoreType.DMA] * NBUF,
)
def _emb_lookup(idx_hbm, table_hbm, out_hbm, idx_v, rows_v, *sems):
    wid = lax.axis_index("s") * NC + lax.axis_index("c")
    base = wid * B_PER_W

    pltpu.sync_copy(idx_hbm.at[pl.ds(base, B_PER_W)], idx_v)

    for b in range(NBUF):
        pltpu.async_copy(
            table_hbm.at[idx_v.at[pl.ds(b * C, C)]], rows_v.at[b], sems[b]
        )

    def group(g, carry):
        for b in range(NBUF):
            i = g * NBUF + b
            pltpu.make_async_copy(
                table_hbm.at[idx_v.at[pl.ds(i * C, C)]], rows_v.at[b], sems[b]
            ).wait()
            pltpu.sync_copy(rows_v.at[b], out_hbm.at[pl.ds(base + i * C, C)])
            nxt = i + NBUF

            @pl.when(nxt < NCHUNK)
            def _():
                pltpu.async_copy(
                    table_hbm.at[idx_v.at[pl.ds(nxt * C, C)]],
                    rows_v.at[b],
                    sems[b],
                )

        return carry

    lax.fori_loop(0, NGROUP, group, 0)


def kernel(token_ids, embedding):
    flat = token_ids.reshape(-1).astype(jnp.int32)
    out = _emb_lookup(flat, embedding)
    return out.reshape(token_ids.shape + (embedding.shape[1],))


# speedup vs baseline: 1.7739x; 1.7739x over previous
import functools

import jax
import jax.numpy as jnp
from jax import lax
from jax.experimental import pallas as pl
from jax.experimental.pallas import tpu as pltpu
from jax.experimental.pallas import tpu_sc as plsc

NTOK = 4096
S = 50
D = 128
NC, NS = 2, 16
NW = NC * NS
TOK_PER_W = NTOK // NW
ROWS_PER_W = TOK_PER_W * S
TCHUNK = 8
CR = TCHUNK * S
G = 80
NG = CR // G
NCHUNK = TOK_PER_W // TCHUNK
NBUF = 2
NGROUP = NCHUNK // NBUF

_mesh = plsc.VectorSubcoreMesh(core_axis_name="c", subcore_axis_name="s")


@functools.partial(
    pl.kernel,
    mesh=_mesh,
    out_type=jax.ShapeDtypeStruct((NTOK, S, D), jnp.float32),
    scratch_types=[
        pltpu.VMEM((ROWS_PER_W,), jnp.int32),
        pltpu.VMEM((NBUF * CR, D), jnp.float32),
        pltpu.SemaphoreType.DMA,
        pltpu.SemaphoreType.DMA,
    ],
)
def _emb_lookup(idx_hbm, table_hbm, out_hbm, idx_v, rows_v, sem0, sem1):
    sems = [sem0, sem1]
    wid = lax.axis_index("s") * NC + lax.axis_index("c")
    row_base = wid * ROWS_PER_W
    tok_base = wid * TOK_PER_W

    pltpu.sync_copy(idx_hbm.at[pl.ds(row_base, ROWS_PER_W)], idx_v)

    def start_chunk(c, b):
        for j in range(NG):
            off = j * G
            pltpu.async_copy(
                table_hbm.at[idx_v.at[pl.ds(c * CR + off, G)]],
                rows_v.at[pl.ds(b * CR + off, G)],
                sems[b],
            )

    for b in range(NBUF):
        start_chunk(b, b)

    def group(g, carry):
        for b in range(NBUF):
            c = g * NBUF + b
            pltpu.make_async_copy(
                table_hbm.at[pl.ds(0, CR)],
                rows_v.at[pl.ds(b * CR, CR)],
                sems[b],
            ).wait()
            for t in range(TCHUNK):
                pltpu.sync_copy(
                    rows_v.at[pl.ds(b * CR + t * S, S)],
                    out_hbm.at[tok_base + c * TCHUNK + t],
                )
            nxt = c + NBUF

            @pl.when(nxt < NCHUNK)
            def _():
                start_chunk(nxt, b)

        return carry

    lax.fori_loop(0, NGROUP, group, 0)


def kernel(token_ids, embedding):
    flat = token_ids.reshape(-1).astype(jnp.int32)
    return _emb_lookup(flat, embedding)
